# baseline (speedup 1.0000x reference)
import functools

import jax
import jax.numpy as jnp
from jax import lax
from jax.experimental import pallas as pl
from jax.experimental.pallas import tpu as pltpu
from jax.experimental.pallas import tpu_sc as plsc

NC = 2
NS = 16
NW = NC * NS


@functools.lru_cache(maxsize=None)
def _make_transpose(V, D):
    nrb = V // 128
    rem = V - nrb * 128
    W = 128 // D
    SB = 4
    VS = SB * 128
    nsb = nrb // SB
    assert nrb % SB == 0
    mesh = plsc.VectorSubcoreMesh(core_axis_name="c", subcore_axis_name="s")

    @functools.partial(
        pl.kernel,
        mesh=mesh,
        out_type=jax.ShapeDtypeStruct((V // W, 128), jnp.float32),
        scratch_types=[
            pltpu.VMEM((32, VS), jnp.float32),
            pltpu.VMEM((32, VS), jnp.float32),
            pltpu.VMEM((VS // W, 128), jnp.float32),
            pltpu.VMEM((VS // W, 128), jnp.float32),
            pltpu.SemaphoreType.DMA,
            pltpu.SemaphoreType.DMA,
            pltpu.SemaphoreType.DMA,
            pltpu.SemaphoreType.DMA,
        ],
        compiler_params=pltpu.CompilerParams(needs_layout_passes=False),
    )
    def transpose_kernel(tabt_hbm, tail_hbm, out_hbm, inA, inB, outA, outB,
                         gsemA, gsemB, psemA, psemB):
        wid = lax.axis_index("s") * NC + lax.axis_index("c")
        nt = (nsb - 1 - wid) // NW + 1
        nhalf = nt // 2

        row_pat0 = lax.iota(jnp.int32, 16)
        row_pat1 = row_pat0 + 16

        def stage(sb, buf, sem):
            for fb in range(4):
                pltpu.async_copy(
                    tabt_hbm.at[pl.ds(fb * 8, 8), pl.ds(sb * VS, VS)],
                    buf.at[pl.ds(fb * 8, 8)],
                    sem,
                )

        def drain_stage(buf, sem):
            for fb in range(4):
                pltpu.make_async_copy(
                    tabt_hbm.at[pl.ds(0, 8), pl.ds(0, VS)],
                    buf.at[pl.ds(fb * 8, 8)],
                    sem,
                ).wait()

        zero16 = jnp.zeros((16,), jnp.int32)
        col_pats = [zero16 + 16 * k + row_pat0 for k in range(8)]

        def shuffle(src, dst, nq):
            def body(q, _):
                c0 = zero16 + q * W
                qb = zero16 + q
                for k in range(8):
                    rows = row_pat1 if (k % 2) else row_pat0
                    vec = plsc.load_gather(src, [rows, c0 + (k // 2)])
                    plsc.store_scatter(dst, [qb, col_pats[k]], vec)
                return _
            lax.fori_loop(0, nq, body, None, unroll=8)

        def put(sb, buf, sem):
            return pltpu.async_copy(
                buf, out_hbm.at[pl.ds(sb * (VS // W), VS // W)], sem)

        def drain_put(buf, sem):
            pltpu.make_async_copy(
                buf, out_hbm.at[pl.ds(0, VS // W)], sem).wait()

        stage(wid, inA, gsemA)

        @pl.when(nt > 1)
        def _():
            stage(wid + NW, inB, gsemB)

        def body(s, _):
            bA = wid + 2 * NW * s

            @pl.when(s > 0)
            def _():
                drain_put(outA, psemA)
            drain_stage(inA, gsemA)
            shuffle(inA, outA, VS // W)

            @pl.when(2 * s + 2 < nt)
            def _():
                stage(bA + 2 * NW, inA, gsemA)
            put(bA, outA, psemA)

            @pl.when(s > 0)
            def _():
                drain_put(outB, psemB)
            drain_stage(inB, gsemB)
            shuffle(inB, outB, VS // W)

            @pl.when(2 * s + 3 < nt)
            def _():
                stage(bA + 3 * NW, inB, gsemB)
            put(bA + NW, outB, psemB)
            return _

        lax.fori_loop(0, nhalf, body, None)

        @pl.when(nt % 2 == 1)
        def _():
            b = wid + (nt - 1) * NW

            @pl.when(nhalf > 0)
            def _():
                drain_put(outA, psemA)
            drain_stage(inA, gsemA)
            shuffle(inA, outA, VS // W)
            put(b, outA, psemA)
            drain_put(outA, psemA)

        @pl.when((nt % 2 == 0) & (nhalf > 0))
        def _():
            drain_put(outA, psemA)

        @pl.when(nhalf > 0)
        def _():
            drain_put(outB, psemB)

        if rem:
            @pl.when(wid == 0)
            def _():
                pltpu.sync_copy(
                    tail_hbm,
                    out_hbm.at[pl.ds(nrb * 32, rem // W)],
                )

    return transpose_kernel


@functools.lru_cache(maxsize=None)
def _make_gather(V, D, B):
    assert B % NW == 0
    b_per_w = B // NW
    CH = 832
    assert b_per_w % CH == 0
    nchunk = b_per_w // CH
    mesh = plsc.VectorSubcoreMesh(core_axis_name="c", subcore_axis_name="s")

    @functools.partial(
        pl.kernel,
        mesh=mesh,
        out_type=jax.ShapeDtypeStruct((B, D), jnp.float32),
        scratch_types=[
            pltpu.VMEM((b_per_w,), jnp.int32),
            pltpu.VMEM((CH, D), jnp.float32),
            pltpu.VMEM((CH, D), jnp.float32),
            pltpu.SemaphoreType.DMA,
            pltpu.SemaphoreType.DMA,
            pltpu.SemaphoreType.DMA,
        ],
        compiler_params=pltpu.CompilerParams(use_tc_tiling_on_sc=False),
    )
    def gather_kernel(table_hbm, idx_hbm, out_hbm, idx_v, rows0, rows1,
                      gsem, psem0, psem1):
        wid = lax.axis_index("s") * NC + lax.axis_index("c")
        base = wid * b_per_w
        pltpu.sync_copy(idx_hbm.at[pl.ds(base, b_per_w)], idx_v)

        bufs = (rows0, rows1)
        psems = (psem0, psem1)

        def start_gather(g):
            return pltpu.async_copy(
                table_hbm.at[idx_v.at[pl.ds(g * CH, CH)]],
                bufs[g % 2],
                gsem,
            )

        puts = [None] * nchunk
        gathers = [None] * (nchunk + 1)
        gathers[0] = start_gather(0)
        for g in range(nchunk):
            gathers[g].wait()
            puts[g] = pltpu.async_copy(
                bufs[g % 2],
                out_hbm.at[pl.ds(base + g * CH, CH)],
                psems[g % 2],
            )
            if g + 1 < nchunk:
                if g >= 1:
                    puts[g - 1].wait()
                gathers[g + 1] = start_gather(g + 1)
        puts[nchunk - 1].wait()
        if nchunk >= 2:
            puts[nchunk - 2].wait()

    return gather_kernel


def kernel(x, table):
    B0, B1 = x.shape
    V, D = table.shape
    B = B0 * B1
    nrb = V // 128
    tail = table[nrb * 128:].reshape((V - nrb * 128) * D // 128, 128)
    scratch = _make_transpose(V, D)(table.T, tail)
    table_lin = scratch.reshape(V, D)
    flat_idx = x.reshape(B)
    out = _make_gather(V, D, B)(table_lin, flat_idx)
    return out.reshape(B0, B1, D)


# speedup vs baseline: 1.3515x; 1.3515x over previous
import functools

import jax
import jax.numpy as jnp
from jax import lax
from jax.experimental import pallas as pl
from jax.experimental.pallas import tpu as pltpu
from jax.experimental.pallas import tpu_sc as plsc

NC = 2
NS = 16
NW = NC * NS


@functools.lru_cache(maxsize=None)
def _make_gather(V, D, B0, B1):
    B = B0 * B1
    assert B % NW == 0 and B0 % NW == 0
    rows_per_w = B0 // NW
    b_per_w = B // NW
    CHR = 32
    CH = CHR * B1
    assert rows_per_w % CHR == 0
    nchunk = rows_per_w // CHR
    mesh = plsc.VectorSubcoreMesh(core_axis_name="c", subcore_axis_name="s")

    @functools.partial(
        pl.kernel,
        mesh=mesh,
        out_type=jax.ShapeDtypeStruct((B, D), jnp.float32),
        scratch_types=[
            pltpu.VMEM((b_per_w,), jnp.int32),
            pltpu.VMEM((CH, D), jnp.float32),
            pltpu.VMEM((CH, D), jnp.float32),
            pltpu.SemaphoreType.DMA,
            pltpu.SemaphoreType.DMA,
            pltpu.SemaphoreType.DMA,
        ],
        compiler_params=pltpu.CompilerParams(use_tc_tiling_on_sc=False),
    )
    def gather_kernel(table_hbm, idx_hbm, out_hbm, idx_v, rows0, rows1,
                      gsem, psem0, psem1):
        wid = lax.axis_index("s") * NC + lax.axis_index("c")
        base = wid * b_per_w
        row0 = wid * rows_per_w
        pltpu.sync_copy(idx_hbm.at[pl.ds(base, b_per_w)], idx_v)

        bufs = (rows0, rows1)
        psems = (psem0, psem1)

        def start_gather(g):
            return pltpu.async_copy(
                table_hbm.at[idx_v.at[pl.ds(g * CH, CH)]],
                bufs[g % 2],
                gsem,
            )

        puts = [None] * nchunk
        gathers = [None] * (nchunk + 1)
        gathers[0] = start_gather(0)
        for g in range(nchunk):
            gathers[g].wait()
            puts[g] = pltpu.async_copy(
                bufs[g % 2],
                out_hbm.at[pl.ds(base + g * CH, CH)],
                psems[g % 2],
            )
            if g + 1 < nchunk:
                if g >= 1:
                    puts[g - 1].wait()
                gathers[g + 1] = start_gather(g + 1)
        puts[nchunk - 1].wait()
        if nchunk >= 2:
            puts[nchunk - 2].wait()

    return gather_kernel


def kernel(x, table):
    B0, B1 = x.shape
    V, D = table.shape
    B = B0 * B1
    flat_idx = x.reshape(B)
    out = _make_gather(V, D, B0, B1)(table, flat_idx)
    return out.reshape(B0, B1, D)
